# Initial kernel scaffold; baseline (speedup 1.0000x reference)
import jax
import jax.numpy as jnp
from jax.experimental import pallas as pl


def kernel(fine_h_A, fine_h_B, coarse_h_A, coarse_h_B, fine_x_A, fine_x_B, coarse_x_A, coarse_x_B, edge_feat_A, edge_feat_B, edge_index_A, edge_index_B, params):
    raise NotImplementedError("write your pallas kernel here")


# speedup vs baseline: 6.7512x; 6.7512x over previous
import functools

import jax
import jax.numpy as jnp
from jax import lax
from jax.experimental import pallas as pl
from jax.experimental.pallas import tpu as pltpu
from jax.experimental.pallas import tpu_sc as plsc

D = 128
DE = 16
NF = 10000
NC = 1000
E = 320000
NCP = 1024
TW = 256
SIGMAS = [1.5 ** x for x in range(15)]
NEG = 0.01
SKIP_H = 0.5

NW = 32
CHUNK = 128
NCHUNK = E // CHUNK
KPW = -(-NCHUNK // NW)
F32 = jnp.float32
BF16 = jnp.bfloat16


def _lrelu(x):
    return jnp.where(x >= 0, x, x * NEG)


def _lnorm(x, g, b):
    m = jnp.mean(x, axis=-1, keepdims=True)
    xc = x - m
    v = jnp.mean(xc * xc, axis=-1, keepdims=True)
    return xc * lax.rsqrt(v + 1e-5) * g + b


U32 = jnp.uint32


def _pack_pair(hi, lo):
    hb = lax.bitcast_convert_type(hi, U32) + jnp.uint32(0x8000)
    lb = lax.bitcast_convert_type(lo, U32) + jnp.uint32(0x8000)
    packed = (hb & jnp.uint32(0xFFFF0000)) | (lb >> 16)
    return lax.bitcast_convert_type(packed, jnp.int32)


def _unpack_hi(p):
    u = lax.bitcast_convert_type(p, U32)
    return lax.bitcast_convert_type(u & jnp.uint32(0xFFFF0000), F32)


def _unpack_lo(p):
    u = lax.bitcast_convert_type(p, U32)
    return lax.bitcast_convert_type(u << 16, F32)


def _fine_tab_body(fa_ref, fb_ref, xa_ref, xb_ref, w_ref, oa_ref, ob_ref):
    w = w_ref[...]

    def pack(h, x):
        proj = jnp.dot(h, w, preferred_element_type=F32)
        xpad = jnp.concatenate(
            [x, jnp.zeros((x.shape[0], D - 16), F32)], axis=1)
        return _pack_pair(proj, xpad)

    oa_ref[...] = pack(fa_ref[...], xa_ref[...])
    ob_ref[...] = pack(fb_ref[...], xb_ref[...])


def _fine_tab(fhA, fhB, fxA, fxB, wf):
    blk = 2000
    return pl.pallas_call(
        _fine_tab_body,
        grid=(NF // blk,),
        in_specs=[pl.BlockSpec((blk, D), lambda i: (i, 0)),
                  pl.BlockSpec((blk, D), lambda i: (i, 0)),
                  pl.BlockSpec((blk, 16), lambda i: (i, 0)),
                  pl.BlockSpec((blk, 16), lambda i: (i, 0)),
                  pl.BlockSpec((D, D), lambda i: (0, 0))],
        out_specs=[pl.BlockSpec((blk, D), lambda i: (i, 0)),
                   pl.BlockSpec((blk, D), lambda i: (i, 0))],
        out_shape=[jax.ShapeDtypeStruct((NF, D), jnp.int32),
                   jax.ShapeDtypeStruct((NF, D), jnp.int32)],
    )(fhA, fhB, fxA, fxB, wf)


def _coarse_att_body(ca_ref, cb_ref, xa_ref, xb_ref, wc_ref, be1_ref,
                     wq_ref, wk_ref, wv_ref,
                     ta_ref, tb_ref, haa_ref, hab_ref):
    ca = ca_ref[...]
    cb = cb_ref[...]
    wc = wc_ref[...]
    be1 = be1_ref[...]

    def pack(h, x):
        proj = jnp.dot(h, wc, preferred_element_type=F32) + be1
        xpad = jnp.concatenate(
            [x, jnp.zeros((x.shape[0], D - 16), F32)], axis=1)
        return _pack_pair(proj, xpad)

    ta_ref[...] = pack(ca, xa_ref[...])
    tb_ref[...] = pack(cb, xb_ref[...])

    wq = wq_ref[...]
    wk = wk_ref[...]
    wv = wv_ref[...]
    qa = _lrelu(jnp.dot(ca, wq, preferred_element_type=F32))
    ka = _lrelu(jnp.dot(ca, wk, preferred_element_type=F32))
    va = jnp.dot(ca, wv, preferred_element_type=F32)
    qb = _lrelu(jnp.dot(cb, wq, preferred_element_type=F32))
    kb = _lrelu(jnp.dot(cb, wk, preferred_element_type=F32))
    vb = jnp.dot(cb, wv, preferred_element_type=F32)

    def att(q, k, v):
        s = lax.dot_general(q, k, (((1,), (1,)), ((), ())),
                            preferred_element_type=F32)
        s = s - jnp.max(s, axis=-1, keepdims=True)
        e = jnp.exp(s)
        num = jnp.dot(e, v, preferred_element_type=F32)
        return num / jnp.sum(e, axis=-1, keepdims=True)

    haa_ref[...] = ca + att(qa, kb, vb)
    hab_ref[...] = cb + att(qb, ka, va)


def _coarse_att(chA, chB, cxA, cxB, wc, be1, wq, wk, wv):
    full = lambda s: pl.BlockSpec(s, lambda: (0,) * len(s))
    return pl.pallas_call(
        _coarse_att_body,
        in_specs=[full((NC, D)), full((NC, D)), full((NC, 16)),
                  full((NC, 16)), full((D, D)), full((1, D)),
                  full((D, D)), full((D, D)), full((D, D))],
        out_specs=[full((NC, D)), full((NC, D)),
                   full((NC, D)), full((NC, D))],
        out_shape=[jax.ShapeDtypeStruct((NC, D), jnp.int32),
                   jax.ShapeDtypeStruct((NC, D), jnp.int32),
                   jax.ShapeDtypeStruct((NC, D), F32),
                   jax.ShapeDtypeStruct((NC, D), F32)],
    )(chA, chB, cxA, cxB, wc, be1, wq, wk, wv)


def _sc_gather(ftab, ctab, src, dst):
    mesh = plsc.VectorSubcoreMesh(core_axis_name="c", subcore_axis_name="s")

    @functools.partial(
        pl.kernel, mesh=mesh,
        out_type=[jax.ShapeDtypeStruct((E, D), jnp.int32),
                  jax.ShapeDtypeStruct((E, D), jnp.int32)],
        scratch_types=[pltpu.VMEM((CHUNK,), jnp.int32),
                       pltpu.VMEM((CHUNK,), jnp.int32),
                       pltpu.VMEM((CHUNK,), jnp.int32),
                       pltpu.VMEM((CHUNK,), jnp.int32),
                       pltpu.VMEM((CHUNK, D), jnp.int32),
                       pltpu.VMEM((CHUNK, D), jnp.int32),
                       pltpu.VMEM((CHUNK, D), jnp.int32),
                       pltpu.VMEM((CHUNK, D), jnp.int32),
                       pltpu.SemaphoreType.DMA,
                       pltpu.SemaphoreType.DMA,
                       pltpu.SemaphoreType.DMA,
                       pltpu.SemaphoreType.DMA],
    )
    def k(ftab_h, ctab_h, src_h, dst_h, g1_h, g2_h,
          src0, src1, dst0, dst1, bf0, bf1, bc0, bc1, gs0, gs1, ws0, ws1):
        wid = lax.axis_index("s") * 2 + lax.axis_index("c")
        srcv = (src0, src1)
        dstv = (dst0, dst1)
        bf = (bf0, bf1)
        bc = (bc0, bc1)
        gsem = (gs0, gs1)
        wsem = (ws0, ws1)

        def cid_of(i):
            return wid + i * NW

        def start(i, s):
            off = cid_of(i) * CHUNK
            pltpu.sync_copy(src_h.at[pl.ds(off, CHUNK)], srcv[s])
            pltpu.sync_copy(dst_h.at[pl.ds(off, CHUNK)], dstv[s])
            c1 = pltpu.async_copy(ftab_h.at[srcv[s]], bf[s], gsem[s])
            c2 = pltpu.async_copy(ctab_h.at[dstv[s]], bc[s], gsem[s])
            return c1, c2

        def write(i, s):
            off = cid_of(i) * CHUNK
            w1 = pltpu.async_copy(bf[s], g1_h.at[pl.ds(off, CHUNK)], wsem[s])
            w2 = pltpu.async_copy(bc[s], g2_h.at[pl.ds(off, CHUNK)], wsem[s])
            return w1, w2

        pend_g = [None, None]
        pend_w = [None, None]
        pend_g[0] = start(0, 0)
        for i in range(KPW):
            s = i & 1
            n = 1 - s
            if i + 1 < KPW:
                if pend_w[n] is not None:
                    for h in pend_w[n]:
                        h.wait()
                    pend_w[n] = None
                if i + 1 == KPW - 1:
                    @pl.when(cid_of(i + 1) < NCHUNK)
                    def _():
                        pend_g[n] = start(i + 1, n)
                else:
                    pend_g[n] = start(i + 1, n)
            if i == KPW - 1:
                @pl.when(cid_of(i) < NCHUNK)
                def _():
                    for h in pend_g[s]:
                        h.wait()
                    for h in write(i, s):
                        h.wait()
            else:
                for h in pend_g[s]:
                    h.wait()
                pend_w[s] = write(i, s)
        for s in range(2):
            if pend_w[s] is not None:
                for h in pend_w[s]:
                    h.wait()

    return k(ftab, ctab, src, dst)


BE = 512


def _edge_body(g1_ref, g2_ref, ef_ref, dstf_ref,
               wef_ref, wrb_ref, sig_ref,
               lg1_ref, lb1_ref, we2_ref, be2_ref, lg2_ref, lb2_ref,
               wc1_ref, bc1_ref, wc2_ref, bc2_ref,
               acc_ref):
    g1 = g1_ref[...]
    g2 = g2_ref[...]
    hf = _unpack_hi(g1)
    cf = _unpack_hi(g2)
    xrel = _unpack_lo(g1[:, :16]) - _unpack_lo(g2[:, :16])
    d2 = jnp.sum(xrel * xrel, axis=-1, keepdims=True)
    rbf = jnp.exp(-d2 * sig_ref[...])
    e1 = hf + cf
    e1 = e1 + jnp.dot(ef_ref[...].astype(BF16), wef_ref[...],
                      preferred_element_type=F32)
    e1 = e1 + jnp.dot(rbf.astype(BF16), wrb_ref[...],
                      preferred_element_type=F32)
    h1 = _lrelu(_lnorm(e1, lg1_ref[...], lb1_ref[...]))
    msg = _lnorm(jnp.dot(h1.astype(BF16), we2_ref[...],
                         preferred_element_type=F32)
                 + be2_ref[...], lg2_ref[...], lb2_ref[...])
    c1 = _lrelu(jnp.dot(msg.astype(BF16), wc1_ref[...],
                        preferred_element_type=F32) + bc1_ref[...])
    coef = jnp.sum(c1 * wc2_ref[...], axis=-1, keepdims=True) + bc2_ref[...]
    lane = lax.broadcasted_iota(jnp.int32, xrel.shape, 1)
    xc16 = jnp.where(lane == 3, 1.0, xrel * coef)
    payload = jnp.concatenate([msg, xc16], axis=1).astype(BF16)
    rows = lax.broadcasted_iota(jnp.int32, (NCP, BE), 0)
    dsti = dstf_ref[...].astype(jnp.int32)
    onehot_t = jnp.where(rows == dsti, 1.0, 0.0).astype(BF16)
    part = jnp.dot(onehot_t, payload, preferred_element_type=F32)

    @pl.when(pl.program_id(0) == 0)
    def _():
        acc_ref[...] = jnp.zeros_like(acc_ref)

    acc_ref[...] += part


def _edge_mlp(g1, g2, ef, dstf, wef, wrb, sig, lg1, lb1, we2, be2,
              lg2, lb2, wc1, bc1, wc2row, bc2):
    rep = lambda s: pl.BlockSpec(s, lambda i: (0,) * len(s))
    return pl.pallas_call(
        _edge_body,
        grid=(E // BE,),
        in_specs=[pl.BlockSpec((BE, D), lambda i: (i, 0)),
                  pl.BlockSpec((BE, D), lambda i: (i, 0)),
                  pl.BlockSpec((BE, DE), lambda i: (i, 0)),
                  pl.BlockSpec((1, BE), lambda i: (0, i)),
                  rep((DE, D)), rep((16, D)), rep((1, 16)),
                  rep((1, D)), rep((1, D)), rep((D, D)), rep((1, D)),
                  rep((1, D)), rep((1, D)), rep((D, D)), rep((1, D)),
                  rep((1, D)), rep((1, 1))],
        out_specs=pl.BlockSpec((NCP, D + 16), lambda i: (0, 0)),
        out_shape=jax.ShapeDtypeStruct((NCP, D + 16), F32),
    )(g1, g2, ef, dstf, wef, wrb, sig, lg1, lb1, we2, be2, lg2, lb2,
      wc1, bc1, wc2row, bc2)


def _finish_body(acca_ref, haa_ref, cha_ref, cxa_ref,
                 accb_ref, hab_ref, chb_ref, cxb_ref,
                 wn1_ref, bn1_ref, gn1_ref, bb1_ref,
                 wn2_ref, bn2_ref, gn2_ref, bb2_ref,
                 ha_out, xa_out, hb_out, xb_out):
    wn1 = wn1_ref[...]
    wn2 = wn2_ref[...]

    def side(acc_ref, ha_ref, ch_ref, cx_ref, h_out, x_out):
        acc = acc_ref[...]
        am = acc[:, :D]
        ax = acc[:, D:]
        cnt = jnp.maximum(ax[:, 3:4], 1.0)
        aggr = am / cnt
        x_out[...] = cx_ref[...] + ax / cnt
        nin = jnp.concatenate([ha_ref[...], aggr], axis=-1)
        h = _lrelu(_lnorm(jnp.dot(nin, wn1, preferred_element_type=F32)
                          + bn1_ref[...], gn1_ref[...], bb1_ref[...]))
        out = _lnorm(jnp.dot(h, wn2, preferred_element_type=F32)
                     + bn2_ref[...], gn2_ref[...], bb2_ref[...])
        h_out[...] = SKIP_H * out + (1.0 - SKIP_H) * ch_ref[...]

    side(acca_ref, haa_ref, cha_ref, cxa_ref, ha_out, xa_out)
    side(accb_ref, hab_ref, chb_ref, cxb_ref, hb_out, xb_out)


def _finish(accA, haA, chA, cxA, accB, haB, chB, cxB,
            wn1, bn1, gn1, bb1, wn2, bn2, gn2, bb2):
    full = lambda s: pl.BlockSpec(s, lambda: (0,) * len(s))
    return pl.pallas_call(
        _finish_body,
        in_specs=[full((NC, D + 16)), full((NC, D)), full((NC, D)),
                  full((NC, 16)),
                  full((NC, D + 16)), full((NC, D)), full((NC, D)),
                  full((NC, 16)),
                  full((2 * D, D)), full((1, D)), full((1, D)), full((1, D)),
                  full((D, D)), full((1, D)), full((1, D)), full((1, D))],
        out_specs=[full((NC, D)), full((NC, 16)),
                   full((NC, D)), full((NC, 16))],
        out_shape=[jax.ShapeDtypeStruct((NC, D), F32),
                   jax.ShapeDtypeStruct((NC, 16), F32),
                   jax.ShapeDtypeStruct((NC, D), F32),
                   jax.ShapeDtypeStruct((NC, 16), F32)],
    )(accA, haA, chA, cxA, accB, haB, chB, cxB,
      wn1, bn1, gn1, bb1, wn2, bn2, gn2, bb2)


def kernel(fine_h_A, fine_h_B, coarse_h_A, coarse_h_B, fine_x_A, fine_x_B,
           coarse_x_A, coarse_x_B, edge_feat_A, edge_feat_B, edge_index_A,
           edge_index_B, params):
    p = params
    We1 = p["We1"]
    wf = We1[:D]
    wc = We1[D:2 * D]
    wef = We1[2 * D:2 * D + DE].astype(BF16)
    wrb = jnp.concatenate([We1[2 * D + DE:], jnp.zeros((1, D), F32)],
                          axis=0).astype(BF16)
    sig = jnp.array([1.0 / s for s in SIGMAS] + [0.0], F32).reshape(1, 16)
    be1 = p["be1"].reshape(1, D)
    row = lambda v: v.reshape(1, -1)

    pad16 = lambda x: jnp.pad(x, ((0, 0), (0, 13)))
    fxA = pad16(fine_x_A)
    fxB = pad16(fine_x_B)
    cxA = pad16(coarse_x_A)
    cxB = pad16(coarse_x_B)

    ftA, ftB = _fine_tab(fine_h_A, fine_h_B, fxA, fxB, wf)
    ctA, ctB, haA, haB = _coarse_att(coarse_h_A, coarse_h_B, cxA, cxB,
                                     wc, be1, p["Wq"], p["Wk"], p["Wv"])

    wc2row = p["Wc2"].reshape(1, D)
    bc2 = p["bc2"].reshape(1, 1)
    we2 = p["We2"].astype(BF16)
    wc1 = p["Wc1"].astype(BF16)

    def side(ftab, ctab, ef, ei):
        g1, g2 = _sc_gather(ftab, ctab, ei[0], ei[1])
        dstf = ei[1].astype(F32).reshape(1, E)
        acc = _edge_mlp(g1, g2, ef, dstf, wef, wrb, sig,
                        row(p["g1"]), row(p["b1"]), we2,
                        row(p["be2"]), row(p["g2"]), row(p["b2"]),
                        wc1, row(p["bc1"]), wc2row, bc2)
        return acc[:NC]

    accA = side(ftA, ctA, edge_feat_A, edge_index_A)
    accB = side(ftB, ctB, edge_feat_B, edge_index_B)

    houtA, xoutA, houtB, xoutB = _finish(
        accA, haA, coarse_h_A, cxA, accB, haB, coarse_h_B, cxB,
        p["Wn1"], row(p["bn1"]), row(p["gn1"]), row(p["bb1"]),
        p["Wn2"], row(p["bn2"]), row(p["gn2"]), row(p["bb2"]))

    return houtA, xoutA[:, :3], houtB, xoutB[:, :3]


# speedup vs baseline: 6.7726x; 1.0032x over previous
import functools

import jax
import jax.numpy as jnp
from jax import lax
from jax.experimental import pallas as pl
from jax.experimental.pallas import tpu as pltpu
from jax.experimental.pallas import tpu_sc as plsc

D = 128
DE = 16
NF = 10000
NC = 1000
E = 320000
NCP = 1024
TW = 256
SIGMAS = [1.5 ** x for x in range(15)]
NEG = 0.01
SKIP_H = 0.5

NW = 32
CHUNK = 128
NCHUNK = E // CHUNK
KPW = -(-NCHUNK // NW)
F32 = jnp.float32
BF16 = jnp.bfloat16


def _lrelu(x):
    return jnp.where(x >= 0, x, x * NEG)


def _lnorm(x, g, b):
    m = jnp.mean(x, axis=-1, keepdims=True)
    xc = x - m
    v = jnp.mean(xc * xc, axis=-1, keepdims=True)
    return xc * lax.rsqrt(v + 1e-5) * g + b


U32 = jnp.uint32


def _pack_pair(hi, lo):
    hb = lax.bitcast_convert_type(hi, U32) + jnp.uint32(0x8000)
    lb = lax.bitcast_convert_type(lo, U32) + jnp.uint32(0x8000)
    packed = (hb & jnp.uint32(0xFFFF0000)) | (lb >> 16)
    return lax.bitcast_convert_type(packed, jnp.int32)


def _unpack_hi(p):
    u = lax.bitcast_convert_type(p, U32)
    return lax.bitcast_convert_type(u & jnp.uint32(0xFFFF0000), F32)


def _unpack_lo(p):
    u = lax.bitcast_convert_type(p, U32)
    return lax.bitcast_convert_type(u << 16, F32)


def _fine_tab_body(fa_ref, fb_ref, xa_ref, xb_ref, w_ref, oa_ref, ob_ref):
    w = w_ref[...]

    def pack(h, x):
        proj = jnp.dot(h, w, preferred_element_type=F32)
        xpad = jnp.concatenate(
            [x, jnp.zeros((x.shape[0], D - 16), F32)], axis=1)
        return _pack_pair(proj, xpad)

    oa_ref[...] = pack(fa_ref[...], xa_ref[...])
    ob_ref[...] = pack(fb_ref[...], xb_ref[...])


def _fine_tab(fhA, fhB, fxA, fxB, wf):
    blk = 2000
    return pl.pallas_call(
        _fine_tab_body,
        grid=(NF // blk,),
        in_specs=[pl.BlockSpec((blk, D), lambda i: (i, 0)),
                  pl.BlockSpec((blk, D), lambda i: (i, 0)),
                  pl.BlockSpec((blk, 16), lambda i: (i, 0)),
                  pl.BlockSpec((blk, 16), lambda i: (i, 0)),
                  pl.BlockSpec((D, D), lambda i: (0, 0))],
        out_specs=[pl.BlockSpec((blk, D), lambda i: (i, 0)),
                   pl.BlockSpec((blk, D), lambda i: (i, 0))],
        out_shape=[jax.ShapeDtypeStruct((NF, D), jnp.int32),
                   jax.ShapeDtypeStruct((NF, D), jnp.int32)],
    )(fhA, fhB, fxA, fxB, wf)


def _coarse_att_body(ca_ref, cb_ref, xa_ref, xb_ref, wc_ref, be1_ref,
                     wq_ref, wk_ref, wv_ref,
                     ta_ref, tb_ref, haa_ref, hab_ref):
    ca = ca_ref[...]
    cb = cb_ref[...]
    wc = wc_ref[...]
    be1 = be1_ref[...]

    def pack(h, x):
        proj = jnp.dot(h, wc, preferred_element_type=F32) + be1
        xpad = jnp.concatenate(
            [x, jnp.zeros((x.shape[0], D - 16), F32)], axis=1)
        return _pack_pair(proj, xpad)

    ta_ref[...] = pack(ca, xa_ref[...])
    tb_ref[...] = pack(cb, xb_ref[...])

    wq = wq_ref[...]
    wk = wk_ref[...]
    wv = wv_ref[...]
    qa = _lrelu(jnp.dot(ca, wq, preferred_element_type=F32))
    ka = _lrelu(jnp.dot(ca, wk, preferred_element_type=F32))
    va = jnp.dot(ca, wv, preferred_element_type=F32)
    qb = _lrelu(jnp.dot(cb, wq, preferred_element_type=F32))
    kb = _lrelu(jnp.dot(cb, wk, preferred_element_type=F32))
    vb = jnp.dot(cb, wv, preferred_element_type=F32)

    def att(q, k, v):
        s = lax.dot_general(q, k, (((1,), (1,)), ((), ())),
                            preferred_element_type=F32)
        s = s - jnp.max(s, axis=-1, keepdims=True)
        e = jnp.exp(s)
        num = jnp.dot(e, v, preferred_element_type=F32)
        return num / jnp.sum(e, axis=-1, keepdims=True)

    haa_ref[...] = ca + att(qa, kb, vb)
    hab_ref[...] = cb + att(qb, ka, va)


def _coarse_att(chA, chB, cxA, cxB, wc, be1, wq, wk, wv):
    full = lambda s: pl.BlockSpec(s, lambda: (0,) * len(s))
    return pl.pallas_call(
        _coarse_att_body,
        in_specs=[full((NC, D)), full((NC, D)), full((NC, 16)),
                  full((NC, 16)), full((D, D)), full((1, D)),
                  full((D, D)), full((D, D)), full((D, D))],
        out_specs=[full((NC, D)), full((NC, D)),
                   full((NC, D)), full((NC, D))],
        out_shape=[jax.ShapeDtypeStruct((NC, D), jnp.int32),
                   jax.ShapeDtypeStruct((NC, D), jnp.int32),
                   jax.ShapeDtypeStruct((NC, D), F32),
                   jax.ShapeDtypeStruct((NC, D), F32)],
    )(chA, chB, cxA, cxB, wc, be1, wq, wk, wv)


def _sc_gather(ftab, ctab, src, dst):
    mesh = plsc.VectorSubcoreMesh(core_axis_name="c", subcore_axis_name="s")

    @functools.partial(
        pl.kernel, mesh=mesh,
        out_type=[jax.ShapeDtypeStruct((E, D), jnp.int32),
                  jax.ShapeDtypeStruct((E, D), jnp.int32)],
        scratch_types=[pltpu.VMEM((CHUNK,), jnp.int32),
                       pltpu.VMEM((CHUNK,), jnp.int32),
                       pltpu.VMEM((CHUNK,), jnp.int32),
                       pltpu.VMEM((CHUNK,), jnp.int32),
                       pltpu.VMEM((CHUNK, D), jnp.int32),
                       pltpu.VMEM((CHUNK, D), jnp.int32),
                       pltpu.VMEM((CHUNK, D), jnp.int32),
                       pltpu.VMEM((CHUNK, D), jnp.int32),
                       pltpu.SemaphoreType.DMA,
                       pltpu.SemaphoreType.DMA,
                       pltpu.SemaphoreType.DMA,
                       pltpu.SemaphoreType.DMA],
    )
    def k(ftab_h, ctab_h, src_h, dst_h, g1_h, g2_h,
          src0, src1, dst0, dst1, bf0, bf1, bc0, bc1, gs0, gs1, ws0, ws1):
        wid = lax.axis_index("s") * 2 + lax.axis_index("c")
        srcv = (src0, src1)
        dstv = (dst0, dst1)
        bf = (bf0, bf1)
        bc = (bc0, bc1)
        gsem = (gs0, gs1)
        wsem = (ws0, ws1)

        def cid_of(i):
            return wid + i * NW

        def start(i, s):
            off = cid_of(i) * CHUNK
            pltpu.sync_copy(src_h.at[pl.ds(off, CHUNK)], srcv[s])
            pltpu.sync_copy(dst_h.at[pl.ds(off, CHUNK)], dstv[s])
            c1 = pltpu.async_copy(ftab_h.at[srcv[s]], bf[s], gsem[s])
            c2 = pltpu.async_copy(ctab_h.at[dstv[s]], bc[s], gsem[s])
            return c1, c2

        def write(i, s):
            off = cid_of(i) * CHUNK
            w1 = pltpu.async_copy(bf[s], g1_h.at[pl.ds(off, CHUNK)], wsem[s])
            w2 = pltpu.async_copy(bc[s], g2_h.at[pl.ds(off, CHUNK)], wsem[s])
            return w1, w2

        pend_g = [None, None]
        pend_w = [None, None]
        pend_g[0] = start(0, 0)
        for i in range(KPW):
            s = i & 1
            n = 1 - s
            if i + 1 < KPW:
                if pend_w[n] is not None:
                    for h in pend_w[n]:
                        h.wait()
                    pend_w[n] = None
                if i + 1 == KPW - 1:
                    @pl.when(cid_of(i + 1) < NCHUNK)
                    def _():
                        pend_g[n] = start(i + 1, n)
                else:
                    pend_g[n] = start(i + 1, n)
            if i == KPW - 1:
                @pl.when(cid_of(i) < NCHUNK)
                def _():
                    for h in pend_g[s]:
                        h.wait()
                    for h in write(i, s):
                        h.wait()
            else:
                for h in pend_g[s]:
                    h.wait()
                pend_w[s] = write(i, s)
        for s in range(2):
            if pend_w[s] is not None:
                for h in pend_w[s]:
                    h.wait()

    return k(ftab, ctab, src, dst)


BE = 512


def _edge_body(g1_ref, g2_ref, ef_ref, dstf_ref,
               wef_ref, wrb_ref, sig_ref,
               lg1_ref, lb1_ref, we2_ref, be2_ref, lg2_ref, lb2_ref,
               wc1_ref, bc1_ref, wc2_ref, bc2_ref,
               acc_ref):
    g1 = g1_ref[...]
    g2 = g2_ref[...]
    hf = _unpack_hi(g1)
    cf = _unpack_hi(g2)
    xrel = _unpack_lo(g1[:, :16]) - _unpack_lo(g2[:, :16])
    d2 = jnp.sum(xrel * xrel, axis=-1, keepdims=True)
    rbf = jnp.exp(-d2 * sig_ref[...])
    e1 = hf + cf
    e1 = e1 + jnp.dot(ef_ref[...].astype(BF16), wef_ref[...],
                      preferred_element_type=F32)
    e1 = e1 + jnp.dot(rbf.astype(BF16), wrb_ref[...],
                      preferred_element_type=F32)
    h1 = _lrelu(_lnorm(e1, lg1_ref[...], lb1_ref[...]))
    msg = _lnorm(jnp.dot(h1.astype(BF16), we2_ref[...],
                         preferred_element_type=F32)
                 + be2_ref[...], lg2_ref[...], lb2_ref[...])
    c1 = _lrelu(jnp.dot(msg.astype(BF16), wc1_ref[...],
                        preferred_element_type=F32) + bc1_ref[...])
    coef = jnp.sum(c1 * wc2_ref[...], axis=-1, keepdims=True) + bc2_ref[...]
    lane = lax.broadcasted_iota(jnp.int32, xrel.shape, 1)
    xc16 = jnp.where(lane == 3, 1.0, xrel * coef)
    payload = jnp.concatenate([msg, xc16], axis=1).astype(BF16)
    rows = lax.broadcasted_iota(jnp.int32, (NCP, BE), 0)
    dsti = dstf_ref[...].astype(jnp.int32)
    onehot_t = jnp.where(rows == dsti, 1.0, 0.0).astype(BF16)
    part = jnp.dot(onehot_t, payload, preferred_element_type=F32)

    @pl.when(pl.program_id(0) == 0)
    def _():
        acc_ref[...] = jnp.zeros_like(acc_ref)

    acc_ref[...] += part


def _edge_mlp(g1, g2, ef, dstf, wef, wrb, sig, lg1, lb1, we2, be2,
              lg2, lb2, wc1, bc1, wc2row, bc2):
    rep = lambda s: pl.BlockSpec(s, lambda i: (0,) * len(s))
    return pl.pallas_call(
        _edge_body,
        grid=(E // BE,),
        in_specs=[pl.BlockSpec((BE, D), lambda i: (i, 0)),
                  pl.BlockSpec((BE, D), lambda i: (i, 0)),
                  pl.BlockSpec((BE, DE), lambda i: (i, 0)),
                  pl.BlockSpec((1, BE), lambda i: (0, i)),
                  rep((DE, D)), rep((16, D)), rep((1, 16)),
                  rep((1, D)), rep((1, D)), rep((D, D)), rep((1, D)),
                  rep((1, D)), rep((1, D)), rep((D, D)), rep((1, D)),
                  rep((1, D)), rep((1, 1))],
        out_specs=pl.BlockSpec((NCP, D + 16), lambda i: (0, 0)),
        out_shape=jax.ShapeDtypeStruct((NCP, D + 16), F32),
    )(g1, g2, ef, dstf, wef, wrb, sig, lg1, lb1, we2, be2, lg2, lb2,
      wc1, bc1, wc2row, bc2)


def _finish_body(acca_ref, haa_ref, cha_ref, cxa_ref,
                 accb_ref, hab_ref, chb_ref, cxb_ref,
                 wn1_ref, bn1_ref, gn1_ref, bb1_ref,
                 wn2_ref, bn2_ref, gn2_ref, bb2_ref,
                 ha_out, xa_out, hb_out, xb_out):
    wn1 = wn1_ref[...]
    wn2 = wn2_ref[...]

    def side(acc_ref, ha_ref, ch_ref, cx_ref, h_out, x_out):
        acc = acc_ref[...]
        am = acc[:, :D]
        ax = acc[:, D:]
        cnt = jnp.maximum(ax[:, 3:4], 1.0)
        aggr = am / cnt
        x_out[...] = cx_ref[...] + ax / cnt
        nin = jnp.concatenate([ha_ref[...], aggr], axis=-1)
        h = _lrelu(_lnorm(jnp.dot(nin, wn1, preferred_element_type=F32)
                          + bn1_ref[...], gn1_ref[...], bb1_ref[...]))
        out = _lnorm(jnp.dot(h, wn2, preferred_element_type=F32)
                     + bn2_ref[...], gn2_ref[...], bb2_ref[...])
        h_out[...] = SKIP_H * out + (1.0 - SKIP_H) * ch_ref[...]

    side(acca_ref, haa_ref, cha_ref, cxa_ref, ha_out, xa_out)
    side(accb_ref, hab_ref, chb_ref, cxb_ref, hb_out, xb_out)


def _finish(accA, haA, chA, cxA, accB, haB, chB, cxB,
            wn1, bn1, gn1, bb1, wn2, bn2, gn2, bb2):
    full = lambda s: pl.BlockSpec(s, lambda: (0,) * len(s))
    return pl.pallas_call(
        _finish_body,
        in_specs=[full((NC, D + 16)), full((NC, D)), full((NC, D)),
                  full((NC, 16)),
                  full((NC, D + 16)), full((NC, D)), full((NC, D)),
                  full((NC, 16)),
                  full((2 * D, D)), full((1, D)), full((1, D)), full((1, D)),
                  full((D, D)), full((1, D)), full((1, D)), full((1, D))],
        out_specs=[full((NC, D)), full((NC, 16)),
                   full((NC, D)), full((NC, 16))],
        out_shape=[jax.ShapeDtypeStruct((NC, D), F32),
                   jax.ShapeDtypeStruct((NC, 16), F32),
                   jax.ShapeDtypeStruct((NC, D), F32),
                   jax.ShapeDtypeStruct((NC, 16), F32)],
    )(accA, haA, chA, cxA, accB, haB, chB, cxB,
      wn1, bn1, gn1, bb1, wn2, bn2, gn2, bb2)


def kernel(fine_h_A, fine_h_B, coarse_h_A, coarse_h_B, fine_x_A, fine_x_B,
           coarse_x_A, coarse_x_B, edge_feat_A, edge_feat_B, edge_index_A,
           edge_index_B, params):
    p = params
    We1 = p["We1"]
    wf = We1[:D]
    wc = We1[D:2 * D]
    wef = We1[2 * D:2 * D + DE].astype(BF16)
    wrb = jnp.concatenate([We1[2 * D + DE:], jnp.zeros((1, D), F32)],
                          axis=0).astype(BF16)
    sig = jnp.array([1.0 / s for s in SIGMAS] + [0.0], F32).reshape(1, 16)
    be1 = p["be1"].reshape(1, D)
    row = lambda v: v.reshape(1, -1)

    pad16 = lambda x: jnp.pad(x, ((0, 0), (0, 13)))
    fxA = pad16(fine_x_A)
    fxB = pad16(fine_x_B)
    cxA = pad16(coarse_x_A)
    cxB = pad16(coarse_x_B)

    ftA, ftB = _fine_tab(fine_h_A, fine_h_B, fxA, fxB, wf)
    ctA, ctB, haA, haB = _coarse_att(coarse_h_A, coarse_h_B, cxA, cxB,
                                     wc, be1, p["Wq"], p["Wk"], p["Wv"])

    wc2row = p["Wc2"].reshape(1, D)
    bc2 = p["bc2"].reshape(1, 1)
    we2 = p["We2"].astype(BF16)
    wc1 = p["Wc1"].astype(BF16)

    gA = _sc_gather(ftA, ctA, edge_index_A[0], edge_index_A[1])
    gB = _sc_gather(ftB, ctB, edge_index_B[0], edge_index_B[1])

    def side(g, ef, ei):
        dstf = ei[1].astype(F32).reshape(1, E)
        acc = _edge_mlp(g[0], g[1], ef, dstf, wef, wrb, sig,
                        row(p["g1"]), row(p["b1"]), we2,
                        row(p["be2"]), row(p["g2"]), row(p["b2"]),
                        wc1, row(p["bc1"]), wc2row, bc2)
        return acc[:NC]

    accA = side(gA, edge_feat_A, edge_index_A)
    accB = side(gB, edge_feat_B, edge_index_B)

    houtA, xoutA, houtB, xoutB = _finish(
        accA, haA, coarse_h_A, cxA, accB, haB, coarse_h_B, cxB,
        p["Wn1"], row(p["bn1"]), row(p["gn1"]), row(p["bb1"]),
        p["Wn2"], row(p["bn2"]), row(p["gn2"]), row(p["bb2"]))

    return houtA, xoutA[:, :3], houtB, xoutB[:, :3]
